# baseline (speedup 1.0000x reference)
import functools

import jax
import jax.numpy as jnp
import numpy as np
from jax import lax
from jax.experimental import pallas as pl
from jax.experimental.pallas import tpu as pltpu
from jax.experimental.pallas import tpu_sc as plsc

N = 10000
E = 320000
D = 128
DE = 16

NC = 2
NS = 16
EPT = E // (NC * NS)
CH = 80
NCHUNK = EPT // CH
NPAD = 10240
RPT = NPAD // NS

def _pack_bf16_words(v):
    u = jax.lax.bitcast_convert_type(v, jnp.int32)
    r = (u + 0x7FFF + ((u >> 16) & 1))
    lo = (r[:, :64] >> 16) & 0xFFFF
    hi = r[:, 64:] & jnp.int32(-65536)
    return hi | lo


_EB = 4000
_NB = E // _EB
_XB = N // 5


def _prep_eh2_body(ea_ref, wedge_ref, wne2_ref, x_ref, w1_ref, bne_ref,
                   eh2_ref, xh_ref):
    i = pl.program_id(0)
    w2e = jnp.dot(
        wedge_ref[...], wne2_ref[...], preferred_element_type=jnp.float32
    )
    eh2_ref[...] = _pack_bf16_words(jnp.dot(
        ea_ref[...], w2e, preferred_element_type=jnp.float32
    ))

    @pl.when(i < 5)
    def _():
        xh_ref[...] = (
            jnp.dot(x_ref[...], w1_ref[...],
                    preferred_element_type=jnp.float32)
            + bne_ref[...][None, :]
        )


def _prep_eh2(edge_attr, W_edge, W2, x, W1, b_ne):
    return pl.pallas_call(
        _prep_eh2_body,
        grid=(_NB,),
        in_specs=[
            pl.BlockSpec((_EB, DE), lambda i: (i, 0)),
            pl.BlockSpec((DE, D), lambda i: (0, 0)),
            pl.BlockSpec((D, D), lambda i: (0, 0)),
            pl.BlockSpec((_XB, D), lambda i: (jnp.minimum(i, 4), 0)),
            pl.BlockSpec((D, D), lambda i: (0, 0)),
            pl.BlockSpec((D,), lambda i: (0,)),
        ],
        out_specs=[
            pl.BlockSpec((_EB, D // 2), lambda i: (i, 0)),
            pl.BlockSpec((_XB, D), lambda i: (jnp.minimum(i, 4), 0)),
        ],
        out_shape=[
            jax.ShapeDtypeStruct((E, D // 2), jnp.int32),
            jax.ShapeDtypeStruct((N, D), jnp.float32),
        ],
    )(edge_attr, W_edge, W2, x, W1, b_ne)


def _sc_body(xh_hbm, eh2_hbm, ei_hbm, msum_hbm, deg_hbm,
             src_b0, src_b1, dst_b0, dst_b1, rows_v0, rows_v1, eh_v0, eh_v1,
             ones_v, zdeg_v, msum_sh, deg_sh,
             idx_sem0, idx_sem1, in_sem0, in_sem1):
    c = lax.axis_index("c")
    s = lax.axis_index("s")
    wid = c * NS + s

    src_b = (src_b0, src_b1)
    dst_b = (dst_b0, dst_b1)
    rows_v = (rows_v0, rows_v1)
    eh_v = (eh_v0, eh_v1)
    idx_sems = (idx_sem0, idx_sem1)
    in_sems = (in_sem0, in_sem1)

    zero16 = jnp.zeros((16,), jnp.float32)
    one16 = jnp.ones((16,), jnp.float32)

    @pl.loop(0, CH)
    def _(r):
        for j in range(8):
            rows_v0[r, pl.ds(j * 16, 16)] = zero16

    @pl.loop(0, 128)
    def _(k):
        zdeg_v[pl.ds(k * 16, 16)] = zero16

    for k in range(CH // 16):
        ones_v[pl.ds(k * 16, 16)] = one16

    for t in range(8):
        pltpu.async_copy(
            rows_v0, msum_sh.at[pl.ds(s * RPT + t * CH, CH)], in_sem0
        )
    for t in range(8):
        pltpu.make_async_copy(
            rows_v0, msum_sh.at[pl.ds(s * RPT + t * CH, CH)], in_sem0
        ).wait()

    @pl.when(s == 0)
    def _():
        for t in range(5):
            pltpu.sync_copy(zdeg_v, deg_sh.at[pl.ds(t * 2048, 2048)])

    plsc.subcore_barrier()

    ebase = wid * EPT
    ebase2 = wid * (EPT // 2)

    def fire_idx(ic, b):
        base = ebase + ic * CH
        pltpu.async_copy(ei_hbm.at[pl.ds(base, CH)], src_b[b], idx_sems[b])
        pltpu.async_copy(ei_hbm.at[pl.ds(E + base, CH)], dst_b[b],
                         idx_sems[b])

    def wait_idx(ic, b):
        base = ebase + ic * CH
        pltpu.make_async_copy(
            ei_hbm.at[pl.ds(base, CH)], src_b[b], idx_sems[b]
        ).wait()
        pltpu.make_async_copy(
            ei_hbm.at[pl.ds(E + base, CH)], dst_b[b], idx_sems[b]
        ).wait()

    def fire_data(ic, b):
        pltpu.async_copy(xh_hbm.at[src_b[b]], rows_v[b], in_sems[b])
        pltpu.async_copy(
            eh2_hbm.at[pl.ds(ebase + ic * CH, CH)], eh_v[b], in_sems[b]
        )

    def wait_data(ic, b):
        pltpu.make_async_copy(
            xh_hbm.at[src_b[b]], rows_v[b], in_sems[b]
        ).wait()
        pltpu.make_async_copy(
            eh2_hbm.at[pl.ds(ebase + ic * CH, CH)], eh_v[b], in_sems[b]
        ).wait()

    himask = jnp.full((16,), -65536, jnp.int32)

    def compute(b):
        rv, ev = rows_v[b], eh_v[b]

        @pl.loop(0, CH, unroll=2)
        def _(r):
            for k in range(8):
                w = ev[r, pl.ds((k % 4) * 16, 16)]
                if k < 4:
                    ehp = lax.bitcast_convert_type(w << 16, jnp.float32)
                else:
                    ehp = lax.bitcast_convert_type(w & himask, jnp.float32)
                sl = pl.ds(k * 16, 16)
                rv[r, sl] = jnp.maximum(rv[r, sl] + ehp, 0.0)

    def process(ic, b, nb):
        @pl.when(ic + 1 < NCHUNK)
        def _():
            wait_idx(ic + 1, nb)
            fire_data(ic + 1, nb)

        wait_data(ic, b)
        compute(b)
        pltpu.sync_copy(rows_v[b], msum_sh.at[dst_b[b]], add=True)
        pltpu.sync_copy(ones_v, deg_sh.at[dst_b[b]], add=True)

        @pl.when(ic + 2 < NCHUNK)
        def _():
            fire_idx(ic + 2, b)

    fire_idx(0, 0)
    fire_idx(1, 1)
    wait_idx(0, 0)
    fire_data(0, 0)

    @pl.loop(0, NCHUNK - 1, step=2)
    def _(i):
        process(i, 0, 1)
        process(i + 1, 1, 0)

    process(NCHUNK - 1, 0, 1)

    plsc.subcore_barrier()

    for t in range(5):
        sl = pl.ds(s * RPT + t * 128, 128)
        pltpu.async_copy(msum_sh.at[sl], msum_hbm.at[c, sl], in_sem0)
    for t in range(5):
        sl = pl.ds(s * RPT + t * 128, 128)
        pltpu.make_async_copy(msum_sh.at[sl], msum_hbm.at[c, sl],
                              in_sem0).wait()

    @pl.when(s == 0)
    def _():
        for t in range(5):
            pltpu.sync_copy(
                deg_sh.at[pl.ds(t * 2048, 2048)],
                deg_hbm.at[pl.ds(c * NPAD + t * 2048, 2048)],
            )


_sc_call = functools.partial(
    pl.kernel,
    out_type=(
        jax.ShapeDtypeStruct((NC, NPAD, D), jnp.float32),
        jax.ShapeDtypeStruct((NC * NPAD,), jnp.float32),
    ),
    mesh=plsc.VectorSubcoreMesh(
        core_axis_name="c", subcore_axis_name="s", num_cores=NC, num_subcores=NS
    ),
    scratch_types=[
        pltpu.VMEM((CH,), jnp.int32),
        pltpu.VMEM((CH,), jnp.int32),
        pltpu.VMEM((CH,), jnp.int32),
        pltpu.VMEM((CH,), jnp.int32),
        pltpu.VMEM((CH, D), jnp.float32),
        pltpu.VMEM((CH, D), jnp.float32),
        pltpu.VMEM((CH, D // 2), jnp.int32),
        pltpu.VMEM((CH, D // 2), jnp.int32),
        pltpu.VMEM((CH,), jnp.float32),
        pltpu.VMEM((2048,), jnp.float32),
        pltpu.VMEM_SHARED((NPAD, D), jnp.float32),
        pltpu.VMEM_SHARED((NPAD,), jnp.float32),
        pltpu.SemaphoreType.DMA,
        pltpu.SemaphoreType.DMA,
        pltpu.SemaphoreType.DMA,
        pltpu.SemaphoreType.DMA,
    ],
)(_sc_body)


def _comb_body(p_ref, deg_ref, x_ref, ws_ref, bs_ref, o_ref):
    ms = p_ref[0] + p_ref[1]
    d = deg_ref[0] + deg_ref[1]
    r = 1.0 / jnp.maximum(d, 1.0)
    sf = (
        jnp.dot(x_ref[...], ws_ref[...], preferred_element_type=jnp.float32)
        + bs_ref[...][None, :]
    )
    o_ref[...] = ms * r + sf


def _combine(msum, deg, x, W_self, b_self):
    nb = 5
    rb = N // nb
    return pl.pallas_call(
        _comb_body,
        grid=(nb,),
        in_specs=[
            pl.BlockSpec((NC, rb, D), lambda i: (0, i, 0)),
            pl.BlockSpec((NC, rb, 1), lambda i: (0, i, 0)),
            pl.BlockSpec((rb, D), lambda i: (i, 0)),
            pl.BlockSpec((D, D), lambda i: (0, 0)),
            pl.BlockSpec((D,), lambda i: (0,)),
        ],
        out_specs=pl.BlockSpec((rb, D), lambda i: (i, 0)),
        out_shape=jax.ShapeDtypeStruct((N, D), jnp.float32),
    )(msum, deg, x, W_self, b_self)


def kernel(x, edge_index, edge_attr, W_edge, W_ne, b_ne, W_self, b_self):
    W1 = W_ne[:D, :]
    W2 = W_ne[D:, :]
    eh2, xh = _prep_eh2(edge_attr, W_edge, W2, x, W1, b_ne)
    ei_flat = edge_index.reshape(2 * E)
    msum, deg = _sc_call(xh, eh2, ei_flat)
    deg2 = deg.reshape(NC, NPAD, 1)
    return _combine(msum, deg2, x, W_self, b_self)


# speedup vs baseline: 1.0003x; 1.0003x over previous
import functools

import jax
import jax.numpy as jnp
import numpy as np
from jax import lax
from jax.experimental import pallas as pl
from jax.experimental.pallas import tpu as pltpu
from jax.experimental.pallas import tpu_sc as plsc

N = 10000
E = 320000
D = 128
DE = 16

NC = 2
NS = 16
EPT = E // (NC * NS)
CH = 80
NCHUNK = EPT // CH
NPAD = 10240
RPT = NPAD // NS

def _pack_bf16_words(v):
    u = jax.lax.bitcast_convert_type(v, jnp.int32)
    r = (u + 0x7FFF + ((u >> 16) & 1))
    lo = (r[:, :64] >> 16) & 0xFFFF
    hi = r[:, 64:] & jnp.int32(-65536)
    return hi | lo


def _prep_body(x_ref, w1_ref, bne_ref, wedge_ref, wne2_ref, xh_ref, w2e_ref):
    i = pl.program_id(0)
    xh_ref[...] = (
        jnp.dot(x_ref[...], w1_ref[...], preferred_element_type=jnp.float32)
        + bne_ref[...][None, :]
    )

    @pl.when(i == 0)
    def _():
        w2e_ref[...] = jnp.dot(
            wedge_ref[...], wne2_ref[...], preferred_element_type=jnp.float32
        )


def _prep(x, W1, b_ne, W_edge, W2):
    nb = 5
    rb = N // nb
    return pl.pallas_call(
        _prep_body,
        grid=(nb,),
        in_specs=[
            pl.BlockSpec((rb, D), lambda i: (i, 0)),
            pl.BlockSpec((D, D), lambda i: (0, 0)),
            pl.BlockSpec((D,), lambda i: (0,)),
            pl.BlockSpec((DE, D), lambda i: (0, 0)),
            pl.BlockSpec((D, D), lambda i: (0, 0)),
        ],
        out_specs=[
            pl.BlockSpec((rb, D), lambda i: (i, 0)),
            pl.BlockSpec((DE, D), lambda i: (0, 0)),
        ],
        out_shape=[
            jax.ShapeDtypeStruct((N, D), jnp.float32),
            jax.ShapeDtypeStruct((DE, D), jnp.float32),
        ],
    )(x, W1, b_ne, W_edge, W2)


def _eh2_body(ea_ref, w2e_ref, eh2_ref):
    eh2_ref[...] = _pack_bf16_words(jnp.dot(
        ea_ref[...], w2e_ref[...], preferred_element_type=jnp.float32
    ))


def _eh2(edge_attr, W2e):
    eb = 4000
    nb = E // eb
    return pl.pallas_call(
        _eh2_body,
        grid=(nb,),
        in_specs=[
            pl.BlockSpec((eb, DE), lambda i: (i, 0)),
            pl.BlockSpec((DE, D), lambda i: (0, 0)),
        ],
        out_specs=pl.BlockSpec((eb, D // 2), lambda i: (i, 0)),
        out_shape=jax.ShapeDtypeStruct((E, D // 2), jnp.int32),
    )(edge_attr, W2e)


def _sc_body(xh_hbm, eh2_hbm, ei_hbm, msum_hbm, deg_hbm,
             src_b0, src_b1, dst_b0, dst_b1, rows_v0, rows_v1, eh_v0, eh_v1,
             ones_v, zdeg_v, msum_sh, deg_sh,
             idx_sem0, idx_sem1, in_sem0, in_sem1):
    c = lax.axis_index("c")
    s = lax.axis_index("s")
    wid = c * NS + s

    src_b = (src_b0, src_b1)
    dst_b = (dst_b0, dst_b1)
    rows_v = (rows_v0, rows_v1)
    eh_v = (eh_v0, eh_v1)
    idx_sems = (idx_sem0, idx_sem1)
    in_sems = (in_sem0, in_sem1)

    zero16 = jnp.zeros((16,), jnp.float32)
    one16 = jnp.ones((16,), jnp.float32)

    @pl.loop(0, CH)
    def _(r):
        for j in range(8):
            rows_v0[r, pl.ds(j * 16, 16)] = zero16

    @pl.loop(0, 128)
    def _(k):
        zdeg_v[pl.ds(k * 16, 16)] = zero16

    for k in range(CH // 16):
        ones_v[pl.ds(k * 16, 16)] = one16

    for t in range(8):
        pltpu.async_copy(
            rows_v0, msum_sh.at[pl.ds(s * RPT + t * CH, CH)], in_sem0
        )
    for t in range(8):
        pltpu.make_async_copy(
            rows_v0, msum_sh.at[pl.ds(s * RPT + t * CH, CH)], in_sem0
        ).wait()

    @pl.when(s == 0)
    def _():
        for t in range(5):
            pltpu.sync_copy(zdeg_v, deg_sh.at[pl.ds(t * 2048, 2048)])

    plsc.subcore_barrier()

    ebase = wid * EPT
    ebase2 = wid * (EPT // 2)

    def fire_idx(ic, b):
        base = ebase + ic * CH
        pltpu.async_copy(ei_hbm.at[pl.ds(base, CH)], src_b[b], idx_sems[b])
        pltpu.async_copy(ei_hbm.at[pl.ds(E + base, CH)], dst_b[b],
                         idx_sems[b])

    def wait_idx(ic, b):
        base = ebase + ic * CH
        pltpu.make_async_copy(
            ei_hbm.at[pl.ds(base, CH)], src_b[b], idx_sems[b]
        ).wait()
        pltpu.make_async_copy(
            ei_hbm.at[pl.ds(E + base, CH)], dst_b[b], idx_sems[b]
        ).wait()

    def fire_data(ic, b):
        pltpu.async_copy(xh_hbm.at[src_b[b]], rows_v[b], in_sems[b])
        pltpu.async_copy(
            eh2_hbm.at[pl.ds(ebase + ic * CH, CH)], eh_v[b], in_sems[b]
        )

    def wait_data(ic, b):
        pltpu.make_async_copy(
            xh_hbm.at[src_b[b]], rows_v[b], in_sems[b]
        ).wait()
        pltpu.make_async_copy(
            eh2_hbm.at[pl.ds(ebase + ic * CH, CH)], eh_v[b], in_sems[b]
        ).wait()

    himask = jnp.full((16,), -65536, jnp.int32)

    def compute(b):
        rv, ev = rows_v[b], eh_v[b]

        @pl.loop(0, CH, unroll=2)
        def _(r):
            for k in range(8):
                w = ev[r, pl.ds((k % 4) * 16, 16)]
                if k < 4:
                    ehp = lax.bitcast_convert_type(w << 16, jnp.float32)
                else:
                    ehp = lax.bitcast_convert_type(w & himask, jnp.float32)
                sl = pl.ds(k * 16, 16)
                rv[r, sl] = jnp.maximum(rv[r, sl] + ehp, 0.0)

    def process(ic, b, nb):
        @pl.when(ic + 1 < NCHUNK)
        def _():
            wait_idx(ic + 1, nb)
            fire_data(ic + 1, nb)

        wait_data(ic, b)
        compute(b)
        pltpu.sync_copy(rows_v[b], msum_sh.at[dst_b[b]], add=True)
        pltpu.sync_copy(ones_v, deg_sh.at[dst_b[b]], add=True)

        @pl.when(ic + 2 < NCHUNK)
        def _():
            fire_idx(ic + 2, b)

    fire_idx(0, 0)
    fire_idx(1, 1)
    wait_idx(0, 0)
    fire_data(0, 0)

    @pl.loop(0, NCHUNK - 1, step=2)
    def _(i):
        process(i, 0, 1)
        process(i + 1, 1, 0)

    process(NCHUNK - 1, 0, 1)

    plsc.subcore_barrier()

    for t in range(5):
        sl = pl.ds(s * RPT + t * 128, 128)
        pltpu.async_copy(msum_sh.at[sl], msum_hbm.at[c, sl], in_sem0)
    for t in range(5):
        sl = pl.ds(s * RPT + t * 128, 128)
        pltpu.make_async_copy(msum_sh.at[sl], msum_hbm.at[c, sl],
                              in_sem0).wait()

    @pl.when(s == 0)
    def _():
        for t in range(5):
            pltpu.sync_copy(
                deg_sh.at[pl.ds(t * 2048, 2048)],
                deg_hbm.at[pl.ds(c * NPAD + t * 2048, 2048)],
            )


_sc_call = functools.partial(
    pl.kernel,
    out_type=(
        jax.ShapeDtypeStruct((NC, NPAD, D), jnp.float32),
        jax.ShapeDtypeStruct((NC * NPAD,), jnp.float32),
    ),
    mesh=plsc.VectorSubcoreMesh(
        core_axis_name="c", subcore_axis_name="s", num_cores=NC, num_subcores=NS
    ),
    scratch_types=[
        pltpu.VMEM((CH,), jnp.int32),
        pltpu.VMEM((CH,), jnp.int32),
        pltpu.VMEM((CH,), jnp.int32),
        pltpu.VMEM((CH,), jnp.int32),
        pltpu.VMEM((CH, D), jnp.float32),
        pltpu.VMEM((CH, D), jnp.float32),
        pltpu.VMEM((CH, D // 2), jnp.int32),
        pltpu.VMEM((CH, D // 2), jnp.int32),
        pltpu.VMEM((CH,), jnp.float32),
        pltpu.VMEM((2048,), jnp.float32),
        pltpu.VMEM_SHARED((NPAD, D), jnp.float32),
        pltpu.VMEM_SHARED((NPAD,), jnp.float32),
        pltpu.SemaphoreType.DMA,
        pltpu.SemaphoreType.DMA,
        pltpu.SemaphoreType.DMA,
        pltpu.SemaphoreType.DMA,
    ],
)(_sc_body)


def _comb_body(p_ref, deg_ref, x_ref, ws_ref, bs_ref, o_ref):
    ms = p_ref[0] + p_ref[1]
    d = deg_ref[0] + deg_ref[1]
    r = 1.0 / jnp.maximum(d, 1.0)
    sf = (
        jnp.dot(x_ref[...], ws_ref[...], preferred_element_type=jnp.float32)
        + bs_ref[...][None, :]
    )
    o_ref[...] = ms * r + sf


def _combine(msum, deg, x, W_self, b_self):
    nb = 5
    rb = N // nb
    return pl.pallas_call(
        _comb_body,
        grid=(nb,),
        in_specs=[
            pl.BlockSpec((NC, rb, D), lambda i: (0, i, 0)),
            pl.BlockSpec((NC, rb, 1), lambda i: (0, i, 0)),
            pl.BlockSpec((rb, D), lambda i: (i, 0)),
            pl.BlockSpec((D, D), lambda i: (0, 0)),
            pl.BlockSpec((D,), lambda i: (0,)),
        ],
        out_specs=pl.BlockSpec((rb, D), lambda i: (i, 0)),
        out_shape=jax.ShapeDtypeStruct((N, D), jnp.float32),
    )(msum, deg, x, W_self, b_self)


def kernel(x, edge_index, edge_attr, W_edge, W_ne, b_ne, W_self, b_self):
    W1 = W_ne[:D, :]
    W2 = W_ne[D:, :]
    xh, W2e = _prep(x, W1, b_ne, W_edge, W2)
    eh2 = _eh2(edge_attr, W2e)
    ei_flat = edge_index.reshape(2 * E)
    msum, deg = _sc_call(xh, eh2, ei_flat)
    deg2 = deg.reshape(NC, NPAD, 1)
    return _combine(msum, deg2, x, W_self, b_self)


# speedup vs baseline: 1.7483x; 1.7477x over previous
import functools

import jax
import jax.numpy as jnp
import numpy as np
from jax import lax
from jax.experimental import pallas as pl
from jax.experimental.pallas import tpu as pltpu
from jax.experimental.pallas import tpu_sc as plsc

N = 10000
E = 320000
D = 128
DE = 16

NC = 2
NS = 16
EPT = E // (NC * NS)
CH = 80
NCHUNK = EPT // CH
NPAD = 10240
RPT = NPAD // NS

def _pack_bf16_words(v):
    u = jax.lax.bitcast_convert_type(v, jnp.int32)
    r = (u + 0x7FFF + ((u >> 16) & 1))
    lo = (r[:, :64] >> 16) & 0xFFFF
    hi = r[:, 64:] & jnp.int32(-65536)
    return hi | lo


def _prep_body(x_ref, w1_ref, bne_ref, wedge_ref, wne2_ref, xh_ref, w2e_ref):
    i = pl.program_id(0)
    xh_ref[...] = (
        jnp.dot(x_ref[...], w1_ref[...], preferred_element_type=jnp.float32)
        + bne_ref[...][None, :]
    )

    @pl.when(i == 0)
    def _():
        w2e_ref[...] = jnp.dot(
            wedge_ref[...], wne2_ref[...], preferred_element_type=jnp.float32
        )


def _prep(x, W1, b_ne, W_edge, W2):
    nb = 5
    rb = N // nb
    return pl.pallas_call(
        _prep_body,
        grid=(nb,),
        in_specs=[
            pl.BlockSpec((rb, D), lambda i: (i, 0)),
            pl.BlockSpec((D, D), lambda i: (0, 0)),
            pl.BlockSpec((D,), lambda i: (0,)),
            pl.BlockSpec((DE, D), lambda i: (0, 0)),
            pl.BlockSpec((D, D), lambda i: (0, 0)),
        ],
        out_specs=[
            pl.BlockSpec((rb, D), lambda i: (i, 0)),
            pl.BlockSpec((DE, D), lambda i: (0, 0)),
        ],
        out_shape=[
            jax.ShapeDtypeStruct((N, D), jnp.float32),
            jax.ShapeDtypeStruct((DE, D), jnp.float32),
        ],
    )(x, W1, b_ne, W_edge, W2)


def _eh2_body(ea_ref, w2e_ref, eh2_ref):
    eh2_ref[...] = _pack_bf16_words(jnp.dot(
        ea_ref[...], w2e_ref[...], preferred_element_type=jnp.float32
    ))


def _eh2(edge_attr, W2e):
    eb = 4000
    nb = E // eb
    return pl.pallas_call(
        _eh2_body,
        grid=(nb,),
        in_specs=[
            pl.BlockSpec((eb, DE), lambda i: (i, 0)),
            pl.BlockSpec((DE, D), lambda i: (0, 0)),
        ],
        out_specs=pl.BlockSpec((eb, D // 2), lambda i: (i, 0)),
        out_shape=jax.ShapeDtypeStruct((E, D // 2), jnp.int32),
    )(edge_attr, W2e)


def _sc_body(xh_hbm, eh2_hbm, ei_hbm, msum_hbm, deg_hbm,
             src_b0, src_b1, dst_b0, dst_b1, rows_v0, rows_v1, eh_v0, eh_v1,
             ones_v, zdeg_v, msum_sh, deg_sh,
             idx_sem0, idx_sem1, in_sem0, in_sem1):
    c = lax.axis_index("c")
    s = lax.axis_index("s")
    wid = c * NS + s

    src_b = (src_b0, src_b1)
    dst_b = (dst_b0, dst_b1)
    rows_v = (rows_v0, rows_v1)
    eh_v = (eh_v0, eh_v1)
    idx_sems = (idx_sem0, idx_sem1)
    in_sems = (in_sem0, in_sem1)

    zero16 = jnp.zeros((16,), jnp.float32)
    one16 = jnp.ones((16,), jnp.float32)

    @pl.loop(0, CH)
    def _(r):
        for j in range(8):
            rows_v0[r, pl.ds(j * 16, 16)] = zero16

    @pl.loop(0, 128)
    def _(k):
        zdeg_v[pl.ds(k * 16, 16)] = zero16

    for k in range(CH // 16):
        ones_v[pl.ds(k * 16, 16)] = one16

    for t in range(8):
        pltpu.async_copy(
            rows_v0, msum_sh.at[pl.ds(s * RPT + t * CH, CH)], in_sem0
        )
    for t in range(8):
        pltpu.make_async_copy(
            rows_v0, msum_sh.at[pl.ds(s * RPT + t * CH, CH)], in_sem0
        ).wait()

    @pl.when(s == 0)
    def _():
        for t in range(5):
            pltpu.sync_copy(zdeg_v, deg_sh.at[pl.ds(t * 2048, 2048)])

    plsc.subcore_barrier()

    ebase = wid * EPT
    ebase2 = wid * (EPT // 2)

    def fire_idx(ic, b):
        base = ebase + ic * CH
        pltpu.async_copy(ei_hbm.at[pl.ds(base, CH)], src_b[b], idx_sems[b])
        pltpu.async_copy(ei_hbm.at[pl.ds(E + base, CH)], dst_b[b],
                         idx_sems[b])

    def wait_idx(ic, b):
        base = ebase + ic * CH
        pltpu.make_async_copy(
            ei_hbm.at[pl.ds(base, CH)], src_b[b], idx_sems[b]
        ).wait()
        pltpu.make_async_copy(
            ei_hbm.at[pl.ds(E + base, CH)], dst_b[b], idx_sems[b]
        ).wait()

    def fire_data(ic, b):
        pltpu.async_copy(xh_hbm.at[src_b[b]], rows_v[b], in_sems[b])
        pltpu.async_copy(
            eh2_hbm.at[pl.ds(ebase + ic * CH, CH)], eh_v[b], in_sems[b]
        )

    def wait_data(ic, b):
        pltpu.make_async_copy(
            xh_hbm.at[src_b[b]], rows_v[b], in_sems[b]
        ).wait()
        pltpu.make_async_copy(
            eh2_hbm.at[pl.ds(ebase + ic * CH, CH)], eh_v[b], in_sems[b]
        ).wait()

    himask = jnp.full((16,), -65536, jnp.int32)

    def compute(b):
        rv, ev = rows_v[b], eh_v[b]

        @pl.loop(0, CH)
        def _(r):
            for k in range(8):
                w = ev[r, pl.ds((k % 4) * 16, 16)]
                if k < 4:
                    ehp = lax.bitcast_convert_type(w << 16, jnp.float32)
                else:
                    ehp = lax.bitcast_convert_type(w & himask, jnp.float32)
                sl = pl.ds(k * 16, 16)
                rv[r, sl] = jnp.maximum(rv[r, sl] + ehp, 0.0)

    def process(ic, b, nb):
        @pl.when(ic + 1 < NCHUNK)
        def _():
            wait_idx(ic + 1, nb)
            fire_data(ic + 1, nb)

        wait_data(ic, b)
        compute(b)
        pltpu.sync_copy(rows_v[b], msum_sh.at[dst_b[b]], add=True)
        pltpu.sync_copy(ones_v, deg_sh.at[dst_b[b]], add=True)

        @pl.when(ic + 2 < NCHUNK)
        def _():
            fire_idx(ic + 2, b)

    fire_idx(0, 0)
    fire_idx(1, 1)
    wait_idx(0, 0)
    fire_data(0, 0)

    @pl.loop(0, NCHUNK - 1, step=2)
    def _(i):
        process(i, 0, 1)
        process(i + 1, 1, 0)

    process(NCHUNK - 1, 0, 1)

    plsc.subcore_barrier()

    for t in range(5):
        sl = pl.ds(s * RPT + t * 128, 128)
        pltpu.async_copy(msum_sh.at[sl], msum_hbm.at[c, sl], in_sem0)
    for t in range(5):
        sl = pl.ds(s * RPT + t * 128, 128)
        pltpu.make_async_copy(msum_sh.at[sl], msum_hbm.at[c, sl],
                              in_sem0).wait()

    @pl.when(s == 0)
    def _():
        for t in range(5):
            pltpu.sync_copy(
                deg_sh.at[pl.ds(t * 2048, 2048)],
                deg_hbm.at[pl.ds(c * NPAD + t * 2048, 2048)],
            )


_sc_call = functools.partial(
    pl.kernel,
    out_type=(
        jax.ShapeDtypeStruct((NC, NPAD, D), jnp.float32),
        jax.ShapeDtypeStruct((NC * NPAD,), jnp.float32),
    ),
    mesh=plsc.VectorSubcoreMesh(
        core_axis_name="c", subcore_axis_name="s", num_cores=NC, num_subcores=NS
    ),
    scratch_types=[
        pltpu.VMEM((CH,), jnp.int32),
        pltpu.VMEM((CH,), jnp.int32),
        pltpu.VMEM((CH,), jnp.int32),
        pltpu.VMEM((CH,), jnp.int32),
        pltpu.VMEM((CH, D), jnp.float32),
        pltpu.VMEM((CH, D), jnp.float32),
        pltpu.VMEM((CH, D // 2), jnp.int32),
        pltpu.VMEM((CH, D // 2), jnp.int32),
        pltpu.VMEM((CH,), jnp.float32),
        pltpu.VMEM((2048,), jnp.float32),
        pltpu.VMEM_SHARED((NPAD, D), jnp.float32),
        pltpu.VMEM_SHARED((NPAD,), jnp.float32),
        pltpu.SemaphoreType.DMA,
        pltpu.SemaphoreType.DMA,
        pltpu.SemaphoreType.DMA,
        pltpu.SemaphoreType.DMA,
    ],
)(_sc_body)


def _comb_body(p_ref, deg_ref, x_ref, ws_ref, bs_ref, o_ref):
    ms = p_ref[0] + p_ref[1]
    d = deg_ref[0] + deg_ref[1]
    r = 1.0 / jnp.maximum(d, 1.0)
    sf = (
        jnp.dot(x_ref[...], ws_ref[...], preferred_element_type=jnp.float32)
        + bs_ref[...][None, :]
    )
    o_ref[...] = ms * r + sf


def _combine(msum, deg, x, W_self, b_self):
    nb = 5
    rb = N // nb
    return pl.pallas_call(
        _comb_body,
        grid=(nb,),
        in_specs=[
            pl.BlockSpec((NC, rb, D), lambda i: (0, i, 0)),
            pl.BlockSpec((NC, rb, 1), lambda i: (0, i, 0)),
            pl.BlockSpec((rb, D), lambda i: (i, 0)),
            pl.BlockSpec((D, D), lambda i: (0, 0)),
            pl.BlockSpec((D,), lambda i: (0,)),
        ],
        out_specs=pl.BlockSpec((rb, D), lambda i: (i, 0)),
        out_shape=jax.ShapeDtypeStruct((N, D), jnp.float32),
    )(msum, deg, x, W_self, b_self)


def kernel(x, edge_index, edge_attr, W_edge, W_ne, b_ne, W_self, b_self):
    W1 = W_ne[:D, :]
    W2 = W_ne[D:, :]
    xh, W2e = _prep(x, W1, b_ne, W_edge, W2)
    eh2 = _eh2(edge_attr, W2e)
    ei_flat = edge_index.reshape(2 * E)
    msum, deg = _sc_call(xh, eh2, ei_flat)
    deg2 = deg.reshape(NC, NPAD, 1)
    return _combine(msum, deg2, x, W_self, b_self)


# speedup vs baseline: 1.8063x; 1.0332x over previous
import functools

import jax
import jax.numpy as jnp
import numpy as np
from jax import lax
from jax.experimental import pallas as pl
from jax.experimental.pallas import tpu as pltpu
from jax.experimental.pallas import tpu_sc as plsc

N = 10000
E = 320000
D = 128
DE = 16

NC = 2
NS = 16
EPT = E // (NC * NS)
CH = 80
NCHUNK = EPT // CH
NPAD = 10240
RPT = NPAD // NS

def _pack_bf16_words(v):
    u = jax.lax.bitcast_convert_type(v, jnp.int32)
    r = (u + 0x7FFF + ((u >> 16) & 1))
    lo = (r[:, :64] >> 16) & 0xFFFF
    hi = r[:, 64:] & jnp.int32(-65536)
    return hi | lo


def _prep_body(x_ref, w1_ref, bne_ref, wedge_ref, wne2_ref, xh_ref, w2e_ref):
    i = pl.program_id(0)
    xh_ref[...] = (
        jnp.dot(x_ref[...], w1_ref[...], preferred_element_type=jnp.float32)
        + bne_ref[...][None, :]
    )

    @pl.when(i == 0)
    def _():
        w2e_ref[...] = jnp.dot(
            wedge_ref[...], wne2_ref[...], preferred_element_type=jnp.float32
        )


def _prep(x, W1, b_ne, W_edge, W2):
    nb = 5
    rb = N // nb
    return pl.pallas_call(
        _prep_body,
        grid=(nb,),
        in_specs=[
            pl.BlockSpec((rb, D), lambda i: (i, 0)),
            pl.BlockSpec((D, D), lambda i: (0, 0)),
            pl.BlockSpec((D,), lambda i: (0,)),
            pl.BlockSpec((DE, D), lambda i: (0, 0)),
            pl.BlockSpec((D, D), lambda i: (0, 0)),
        ],
        out_specs=[
            pl.BlockSpec((rb, D), lambda i: (i, 0)),
            pl.BlockSpec((DE, D), lambda i: (0, 0)),
        ],
        out_shape=[
            jax.ShapeDtypeStruct((N, D), jnp.float32),
            jax.ShapeDtypeStruct((DE, D), jnp.float32),
        ],
    )(x, W1, b_ne, W_edge, W2)


def _eh2_body(ea_ref, w2e_ref, eh2_ref):
    eh2_ref[...] = jnp.dot(
        ea_ref[...], w2e_ref[...], preferred_element_type=jnp.float32
    )


def _eh2(edge_attr, W2e):
    eb = 4000
    nb = E // eb
    return pl.pallas_call(
        _eh2_body,
        grid=(nb,),
        in_specs=[
            pl.BlockSpec((eb, DE), lambda i: (i, 0)),
            pl.BlockSpec((DE, D), lambda i: (0, 0)),
        ],
        out_specs=pl.BlockSpec((eb, D), lambda i: (i, 0)),
        out_shape=jax.ShapeDtypeStruct((E, D), jnp.float32),
    )(edge_attr, W2e)


def _sc_body(xh_hbm, eh2_hbm, ei_hbm, msum_hbm, deg_hbm,
             src_b0, src_b1, dst_b0, dst_b1, rows_v0, rows_v1, eh_v0, eh_v1,
             ones_v, zdeg_v, msum_sh, deg_sh,
             idx_sem0, idx_sem1, in_sem0, in_sem1):
    c = lax.axis_index("c")
    s = lax.axis_index("s")
    wid = c * NS + s

    src_b = (src_b0, src_b1)
    dst_b = (dst_b0, dst_b1)
    rows_v = (rows_v0, rows_v1)
    eh_v = (eh_v0, eh_v1)
    idx_sems = (idx_sem0, idx_sem1)
    in_sems = (in_sem0, in_sem1)

    zero16 = jnp.zeros((16,), jnp.float32)
    one16 = jnp.ones((16,), jnp.float32)

    @pl.loop(0, CH)
    def _(r):
        for j in range(8):
            rows_v0[r, pl.ds(j * 16, 16)] = zero16

    @pl.loop(0, 128)
    def _(k):
        zdeg_v[pl.ds(k * 16, 16)] = zero16

    for k in range(CH // 16):
        ones_v[pl.ds(k * 16, 16)] = one16

    for t in range(8):
        pltpu.async_copy(
            rows_v0, msum_sh.at[pl.ds(s * RPT + t * CH, CH)], in_sem0
        )
    for t in range(8):
        pltpu.make_async_copy(
            rows_v0, msum_sh.at[pl.ds(s * RPT + t * CH, CH)], in_sem0
        ).wait()

    @pl.when(s == 0)
    def _():
        for t in range(5):
            pltpu.sync_copy(zdeg_v, deg_sh.at[pl.ds(t * 2048, 2048)])

    plsc.subcore_barrier()

    ebase = wid * EPT
    ebase2 = wid * (EPT // 2)

    def fire_idx(ic, b):
        base = ebase + ic * CH
        pltpu.async_copy(ei_hbm.at[pl.ds(base, CH)], src_b[b], idx_sems[b])
        pltpu.async_copy(ei_hbm.at[pl.ds(E + base, CH)], dst_b[b],
                         idx_sems[b])

    def wait_idx(ic, b):
        base = ebase + ic * CH
        pltpu.make_async_copy(
            ei_hbm.at[pl.ds(base, CH)], src_b[b], idx_sems[b]
        ).wait()
        pltpu.make_async_copy(
            ei_hbm.at[pl.ds(E + base, CH)], dst_b[b], idx_sems[b]
        ).wait()

    def fire_data(ic, b):
        pltpu.async_copy(xh_hbm.at[src_b[b]], rows_v[b], in_sems[b])
        pltpu.async_copy(
            eh2_hbm.at[pl.ds(ebase + ic * CH, CH)], eh_v[b], in_sems[b]
        )

    def wait_data(ic, b):
        pltpu.make_async_copy(
            xh_hbm.at[src_b[b]], rows_v[b], in_sems[b]
        ).wait()
        pltpu.make_async_copy(
            eh2_hbm.at[pl.ds(ebase + ic * CH, CH)], eh_v[b], in_sems[b]
        ).wait()

    def compute(b):
        rv, ev = rows_v[b], eh_v[b]

        @pl.loop(0, CH)
        def _(r):
            for k in range(8):
                sl = pl.ds(k * 16, 16)
                rv[r, sl] = jnp.maximum(rv[r, sl] + ev[r, sl], 0.0)

    def process(ic, b, nb):
        @pl.when(ic + 1 < NCHUNK)
        def _():
            wait_idx(ic + 1, nb)
            fire_data(ic + 1, nb)

        wait_data(ic, b)
        compute(b)
        pltpu.sync_copy(rows_v[b], msum_sh.at[dst_b[b]], add=True)
        pltpu.sync_copy(ones_v, deg_sh.at[dst_b[b]], add=True)

        @pl.when(ic + 2 < NCHUNK)
        def _():
            fire_idx(ic + 2, b)

    fire_idx(0, 0)
    fire_idx(1, 1)
    wait_idx(0, 0)
    fire_data(0, 0)

    @pl.loop(0, NCHUNK - 1, step=2)
    def _(i):
        process(i, 0, 1)
        process(i + 1, 1, 0)

    process(NCHUNK - 1, 0, 1)

    plsc.subcore_barrier()

    for t in range(5):
        sl = pl.ds(s * RPT + t * 128, 128)
        pltpu.async_copy(msum_sh.at[sl], msum_hbm.at[c, sl], in_sem0)
    for t in range(5):
        sl = pl.ds(s * RPT + t * 128, 128)
        pltpu.make_async_copy(msum_sh.at[sl], msum_hbm.at[c, sl],
                              in_sem0).wait()

    @pl.when(s == 0)
    def _():
        for t in range(5):
            pltpu.sync_copy(
                deg_sh.at[pl.ds(t * 2048, 2048)],
                deg_hbm.at[pl.ds(c * NPAD + t * 2048, 2048)],
            )


_sc_call = functools.partial(
    pl.kernel,
    out_type=(
        jax.ShapeDtypeStruct((NC, NPAD, D), jnp.float32),
        jax.ShapeDtypeStruct((NC * NPAD,), jnp.float32),
    ),
    mesh=plsc.VectorSubcoreMesh(
        core_axis_name="c", subcore_axis_name="s", num_cores=NC, num_subcores=NS
    ),
    scratch_types=[
        pltpu.VMEM((CH,), jnp.int32),
        pltpu.VMEM((CH,), jnp.int32),
        pltpu.VMEM((CH,), jnp.int32),
        pltpu.VMEM((CH,), jnp.int32),
        pltpu.VMEM((CH, D), jnp.float32),
        pltpu.VMEM((CH, D), jnp.float32),
        pltpu.VMEM((CH, D), jnp.float32),
        pltpu.VMEM((CH, D), jnp.float32),
        pltpu.VMEM((CH,), jnp.float32),
        pltpu.VMEM((2048,), jnp.float32),
        pltpu.VMEM_SHARED((NPAD, D), jnp.float32),
        pltpu.VMEM_SHARED((NPAD,), jnp.float32),
        pltpu.SemaphoreType.DMA,
        pltpu.SemaphoreType.DMA,
        pltpu.SemaphoreType.DMA,
        pltpu.SemaphoreType.DMA,
    ],
)(_sc_body)


def _comb_body(p_ref, deg_ref, x_ref, ws_ref, bs_ref, o_ref):
    ms = p_ref[0] + p_ref[1]
    d = deg_ref[0] + deg_ref[1]
    r = 1.0 / jnp.maximum(d, 1.0)
    sf = (
        jnp.dot(x_ref[...], ws_ref[...], preferred_element_type=jnp.float32)
        + bs_ref[...][None, :]
    )
    o_ref[...] = ms * r + sf


def _combine(msum, deg, x, W_self, b_self):
    nb = 5
    rb = N // nb
    return pl.pallas_call(
        _comb_body,
        grid=(nb,),
        in_specs=[
            pl.BlockSpec((NC, rb, D), lambda i: (0, i, 0)),
            pl.BlockSpec((NC, rb, 1), lambda i: (0, i, 0)),
            pl.BlockSpec((rb, D), lambda i: (i, 0)),
            pl.BlockSpec((D, D), lambda i: (0, 0)),
            pl.BlockSpec((D,), lambda i: (0,)),
        ],
        out_specs=pl.BlockSpec((rb, D), lambda i: (i, 0)),
        out_shape=jax.ShapeDtypeStruct((N, D), jnp.float32),
    )(msum, deg, x, W_self, b_self)


def kernel(x, edge_index, edge_attr, W_edge, W_ne, b_ne, W_self, b_self):
    W1 = W_ne[:D, :]
    W2 = W_ne[D:, :]
    xh, W2e = _prep(x, W1, b_ne, W_edge, W2)
    eh2 = _eh2(edge_attr, W2e)
    ei_flat = edge_index.reshape(2 * E)
    msum, deg = _sc_call(xh, eh2, ei_flat)
    deg2 = deg.reshape(NC, NPAD, 1)
    return _combine(msum, deg2, x, W_self, b_self)


# speedup vs baseline: 1.8337x; 1.0152x over previous
import functools

import jax
import jax.numpy as jnp
import numpy as np
from jax import lax
from jax.experimental import pallas as pl
from jax.experimental.pallas import tpu as pltpu
from jax.experimental.pallas import tpu_sc as plsc

N = 10000
E = 320000
D = 128
DE = 16

NC = 2
NS = 16
EPT = E // (NC * NS)
CH = 80
NCHUNK = EPT // CH
NPAD = 10240
RPT = NPAD // NS

def _pack_bf16_words(v):
    u = jax.lax.bitcast_convert_type(v, jnp.int32)
    r = (u + 0x7FFF + ((u >> 16) & 1))
    lo = (r[:, :64] >> 16) & 0xFFFF
    hi = r[:, 64:] & jnp.int32(-65536)
    return hi | lo


def _prep_body(x_ref, w1_ref, bne_ref, wedge_ref, wne2_ref, xh_ref, w2e_ref):
    i = pl.program_id(0)
    xh_ref[...] = (
        jnp.dot(x_ref[...], w1_ref[...], preferred_element_type=jnp.float32)
        + bne_ref[...][None, :]
    )

    @pl.when(i == 0)
    def _():
        w2e_ref[...] = jnp.dot(
            wedge_ref[...], wne2_ref[...], preferred_element_type=jnp.float32
        )


def _prep(x, W1, b_ne, W_edge, W2):
    nb = 5
    rb = N // nb
    return pl.pallas_call(
        _prep_body,
        grid=(nb,),
        in_specs=[
            pl.BlockSpec((rb, D), lambda i: (i, 0)),
            pl.BlockSpec((D, D), lambda i: (0, 0)),
            pl.BlockSpec((D,), lambda i: (0,)),
            pl.BlockSpec((DE, D), lambda i: (0, 0)),
            pl.BlockSpec((D, D), lambda i: (0, 0)),
        ],
        out_specs=[
            pl.BlockSpec((rb, D), lambda i: (i, 0)),
            pl.BlockSpec((DE, D), lambda i: (0, 0)),
        ],
        out_shape=[
            jax.ShapeDtypeStruct((N, D), jnp.float32),
            jax.ShapeDtypeStruct((DE, D), jnp.float32),
        ],
    )(x, W1, b_ne, W_edge, W2)


def _eh2_body(ea_ref, w2e_ref, eh2_ref):
    eh2_ref[...] = jnp.dot(
        ea_ref[...], w2e_ref[...], preferred_element_type=jnp.float32
    )


def _eh2(edge_attr, W2e):
    eb = 4000
    nb = E // eb
    return pl.pallas_call(
        _eh2_body,
        grid=(nb,),
        in_specs=[
            pl.BlockSpec((eb, DE), lambda i: (i, 0)),
            pl.BlockSpec((DE, D), lambda i: (0, 0)),
        ],
        out_specs=pl.BlockSpec((eb, D), lambda i: (i, 0)),
        out_shape=jax.ShapeDtypeStruct((E, D), jnp.float32),
    )(edge_attr, W2e)


def _sc_body(xh_hbm, eh2_hbm, ei_hbm, msum_hbm, deg_hbm,
             src_b0, src_b1, dst_b0, dst_b1, rows_v0, rows_v1, eh_v0, eh_v1,
             ones_v, zdeg_v, msum_sh, deg_sh,
             idx_sem0, idx_sem1, in_sem0, in_sem1, out_sem):
    c = lax.axis_index("c")
    s = lax.axis_index("s")
    wid = c * NS + s

    src_b = (src_b0, src_b1)
    dst_b = (dst_b0, dst_b1)
    rows_v = (rows_v0, rows_v1)
    eh_v = (eh_v0, eh_v1)
    idx_sems = (idx_sem0, idx_sem1)
    in_sems = (in_sem0, in_sem1)

    zero16 = jnp.zeros((16,), jnp.float32)
    one16 = jnp.ones((16,), jnp.float32)

    @pl.loop(0, CH)
    def _(r):
        for j in range(8):
            rows_v0[r, pl.ds(j * 16, 16)] = zero16

    @pl.loop(0, 128)
    def _(k):
        zdeg_v[pl.ds(k * 16, 16)] = zero16

    for k in range(CH // 16):
        ones_v[pl.ds(k * 16, 16)] = one16

    for t in range(8):
        pltpu.async_copy(
            rows_v0, msum_sh.at[pl.ds(s * RPT + t * CH, CH)], in_sem0
        )
    for t in range(8):
        pltpu.make_async_copy(
            rows_v0, msum_sh.at[pl.ds(s * RPT + t * CH, CH)], in_sem0
        ).wait()

    @pl.when(s == 0)
    def _():
        for t in range(5):
            pltpu.sync_copy(zdeg_v, deg_sh.at[pl.ds(t * 2048, 2048)])

    plsc.subcore_barrier()

    ebase = wid * EPT
    ebase2 = wid * (EPT // 2)

    def fire_idx(ic, b):
        base = ebase + ic * CH
        pltpu.async_copy(ei_hbm.at[pl.ds(base, CH)], src_b[b], idx_sems[b])
        pltpu.async_copy(ei_hbm.at[pl.ds(E + base, CH)], dst_b[b],
                         idx_sems[b])

    def wait_idx(ic, b):
        base = ebase + ic * CH
        pltpu.make_async_copy(
            ei_hbm.at[pl.ds(base, CH)], src_b[b], idx_sems[b]
        ).wait()
        pltpu.make_async_copy(
            ei_hbm.at[pl.ds(E + base, CH)], dst_b[b], idx_sems[b]
        ).wait()

    def fire_data(ic, b):
        pltpu.async_copy(xh_hbm.at[src_b[b]], rows_v[b], in_sems[b])
        pltpu.async_copy(
            eh2_hbm.at[pl.ds(ebase + ic * CH, CH)], eh_v[b], in_sems[b]
        )

    def wait_data(ic, b):
        pltpu.make_async_copy(
            xh_hbm.at[src_b[b]], rows_v[b], in_sems[b]
        ).wait()
        pltpu.make_async_copy(
            eh2_hbm.at[pl.ds(ebase + ic * CH, CH)], eh_v[b], in_sems[b]
        ).wait()

    def compute(b):
        rv, ev = rows_v[b], eh_v[b]

        @pl.loop(0, CH)
        def _(r):
            for k in range(8):
                sl = pl.ds(k * 16, 16)
                rv[r, sl] = jnp.maximum(rv[r, sl] + ev[r, sl], 0.0)

    def process(ic, b, nb):
        @pl.when(ic + 1 < NCHUNK)
        def _():
            wait_idx(ic + 1, nb)
            fire_data(ic + 1, nb)

        wait_data(ic, b)
        compute(b)
        d1 = pltpu.async_copy(rows_v[b], msum_sh.at[dst_b[b]], out_sem,
                              add=True)
        d2 = pltpu.async_copy(ones_v, deg_sh.at[dst_b[b]], out_sem,
                              add=True)
        d1.wait()
        d2.wait()

        @pl.when(ic + 2 < NCHUNK)
        def _():
            fire_idx(ic + 2, b)

    fire_idx(0, 0)
    fire_idx(1, 1)
    wait_idx(0, 0)
    fire_data(0, 0)

    @pl.loop(0, NCHUNK - 1, step=2)
    def _(i):
        process(i, 0, 1)
        process(i + 1, 1, 0)

    process(NCHUNK - 1, 0, 1)

    plsc.subcore_barrier()

    for t in range(5):
        sl = pl.ds(s * RPT + t * 128, 128)
        pltpu.async_copy(msum_sh.at[sl], msum_hbm.at[c, sl], in_sem0)
    for t in range(5):
        sl = pl.ds(s * RPT + t * 128, 128)
        pltpu.make_async_copy(msum_sh.at[sl], msum_hbm.at[c, sl],
                              in_sem0).wait()

    @pl.when(s == 0)
    def _():
        for t in range(5):
            pltpu.sync_copy(
                deg_sh.at[pl.ds(t * 2048, 2048)],
                deg_hbm.at[pl.ds(c * NPAD + t * 2048, 2048)],
            )


_sc_call = functools.partial(
    pl.kernel,
    out_type=(
        jax.ShapeDtypeStruct((NC, NPAD, D), jnp.float32),
        jax.ShapeDtypeStruct((NC * NPAD,), jnp.float32),
    ),
    mesh=plsc.VectorSubcoreMesh(
        core_axis_name="c", subcore_axis_name="s", num_cores=NC, num_subcores=NS
    ),
    scratch_types=[
        pltpu.VMEM((CH,), jnp.int32),
        pltpu.VMEM((CH,), jnp.int32),
        pltpu.VMEM((CH,), jnp.int32),
        pltpu.VMEM((CH,), jnp.int32),
        pltpu.VMEM((CH, D), jnp.float32),
        pltpu.VMEM((CH, D), jnp.float32),
        pltpu.VMEM((CH, D), jnp.float32),
        pltpu.VMEM((CH, D), jnp.float32),
        pltpu.VMEM((CH,), jnp.float32),
        pltpu.VMEM((2048,), jnp.float32),
        pltpu.VMEM_SHARED((NPAD, D), jnp.float32),
        pltpu.VMEM_SHARED((NPAD,), jnp.float32),
        pltpu.SemaphoreType.DMA,
        pltpu.SemaphoreType.DMA,
        pltpu.SemaphoreType.DMA,
        pltpu.SemaphoreType.DMA,
        pltpu.SemaphoreType.DMA,
    ],
)(_sc_body)


def _comb_body(p_ref, deg_ref, x_ref, ws_ref, bs_ref, o_ref):
    ms = p_ref[0] + p_ref[1]
    d = deg_ref[0] + deg_ref[1]
    r = 1.0 / jnp.maximum(d, 1.0)
    sf = (
        jnp.dot(x_ref[...], ws_ref[...], preferred_element_type=jnp.float32)
        + bs_ref[...][None, :]
    )
    o_ref[...] = ms * r + sf


def _combine(msum, deg, x, W_self, b_self):
    nb = 5
    rb = N // nb
    return pl.pallas_call(
        _comb_body,
        grid=(nb,),
        in_specs=[
            pl.BlockSpec((NC, rb, D), lambda i: (0, i, 0)),
            pl.BlockSpec((NC, rb, 1), lambda i: (0, i, 0)),
            pl.BlockSpec((rb, D), lambda i: (i, 0)),
            pl.BlockSpec((D, D), lambda i: (0, 0)),
            pl.BlockSpec((D,), lambda i: (0,)),
        ],
        out_specs=pl.BlockSpec((rb, D), lambda i: (i, 0)),
        out_shape=jax.ShapeDtypeStruct((N, D), jnp.float32),
    )(msum, deg, x, W_self, b_self)


def kernel(x, edge_index, edge_attr, W_edge, W_ne, b_ne, W_self, b_self):
    W1 = W_ne[:D, :]
    W2 = W_ne[D:, :]
    xh, W2e = _prep(x, W1, b_ne, W_edge, W2)
    eh2 = _eh2(edge_attr, W2e)
    ei_flat = edge_index.reshape(2 * E)
    msum, deg = _sc_call(xh, eh2, ei_flat)
    deg2 = deg.reshape(NC, NPAD, 1)
    return _combine(msum, deg2, x, W_self, b_self)


# speedup vs baseline: 1.8361x; 1.0013x over previous
import functools

import jax
import jax.numpy as jnp
import numpy as np
from jax import lax
from jax.experimental import pallas as pl
from jax.experimental.pallas import tpu as pltpu
from jax.experimental.pallas import tpu_sc as plsc

N = 10000
E = 320000
D = 128
DE = 16

NC = 2
NS = 16
EPT = E // (NC * NS)
CH = 80
NCHUNK = EPT // CH
NPAD = 10240
RPT = NPAD // NS

def _pack_bf16_words(v):
    u = jax.lax.bitcast_convert_type(v, jnp.int32)
    r = (u + 0x7FFF + ((u >> 16) & 1))
    lo = (r[:, :64] >> 16) & 0xFFFF
    hi = r[:, 64:] & jnp.int32(-65536)
    return hi | lo


def _prep_body(x_ref, w1_ref, bne_ref, wedge_ref, wne2_ref, xh_ref, w2e_ref):
    i = pl.program_id(0)
    xh_ref[...] = (
        jnp.dot(x_ref[...], w1_ref[...], preferred_element_type=jnp.float32)
        + bne_ref[...][None, :]
    )

    @pl.when(i == 0)
    def _():
        w2e_ref[...] = jnp.dot(
            wedge_ref[...], wne2_ref[...], preferred_element_type=jnp.float32
        )


def _prep(x, W1, b_ne, W_edge, W2):
    nb = 5
    rb = N // nb
    return pl.pallas_call(
        _prep_body,
        grid=(nb,),
        in_specs=[
            pl.BlockSpec((rb, D), lambda i: (i, 0)),
            pl.BlockSpec((D, D), lambda i: (0, 0)),
            pl.BlockSpec((D,), lambda i: (0,)),
            pl.BlockSpec((DE, D), lambda i: (0, 0)),
            pl.BlockSpec((D, D), lambda i: (0, 0)),
        ],
        out_specs=[
            pl.BlockSpec((rb, D), lambda i: (i, 0)),
            pl.BlockSpec((DE, D), lambda i: (0, 0)),
        ],
        out_shape=[
            jax.ShapeDtypeStruct((N, D), jnp.float32),
            jax.ShapeDtypeStruct((DE, D), jnp.float32),
        ],
    )(x, W1, b_ne, W_edge, W2)


def _eh2_body(ea_ref, w2e_ref, eh2_ref):
    eh2_ref[...] = jnp.dot(
        ea_ref[...], w2e_ref[...], preferred_element_type=jnp.float32
    )


def _eh2(edge_attr, W2e):
    eb = 4000
    nb = E // eb
    return pl.pallas_call(
        _eh2_body,
        grid=(nb,),
        in_specs=[
            pl.BlockSpec((eb, DE), lambda i: (i, 0)),
            pl.BlockSpec((DE, D), lambda i: (0, 0)),
        ],
        out_specs=pl.BlockSpec((eb, D), lambda i: (i, 0)),
        out_shape=jax.ShapeDtypeStruct((E, D), jnp.float32),
    )(edge_attr, W2e)


def _sc_body(xh_hbm, eh2_hbm, ei_hbm, msum_hbm, deg_hbm,
             src_b0, src_b1, dst_b0, dst_b1, rows_v0, rows_v1, eh_v0, eh_v1,
             ones_v, zdeg_v, msum_sh, deg_sh,
             idx_sem0, idx_sem1, in_sem0, in_sem1, out_sem):
    c = lax.axis_index("c")
    s = lax.axis_index("s")
    wid = c * NS + s

    src_b = (src_b0, src_b1)
    dst_b = (dst_b0, dst_b1)
    rows_v = (rows_v0, rows_v1)
    eh_v = (eh_v0, eh_v1)
    idx_sems = (idx_sem0, idx_sem1)
    in_sems = (in_sem0, in_sem1)

    zero16 = jnp.zeros((16,), jnp.float32)
    one16 = jnp.ones((16,), jnp.float32)

    @pl.loop(0, CH)
    def _(r):
        for j in range(8):
            rows_v0[r, pl.ds(j * 16, 16)] = zero16

    @pl.loop(0, 128)
    def _(k):
        zdeg_v[pl.ds(k * 16, 16)] = zero16

    for k in range(CH // 16):
        ones_v[pl.ds(k * 16, 16)] = one16

    for t in range(8):
        pltpu.async_copy(
            rows_v0, msum_sh.at[pl.ds(s * RPT + t * CH, CH)], in_sem0
        )
    for t in range(8):
        pltpu.make_async_copy(
            rows_v0, msum_sh.at[pl.ds(s * RPT + t * CH, CH)], in_sem0
        ).wait()

    @pl.when(s == 0)
    def _():
        for t in range(5):
            pltpu.sync_copy(zdeg_v, deg_sh.at[pl.ds(t * 2048, 2048)])

    plsc.subcore_barrier()

    ebase = wid * EPT
    ebase2 = wid * (EPT // 2)

    def fire_idx(ic, b):
        base = ebase + ic * CH
        pltpu.async_copy(ei_hbm.at[pl.ds(base, CH)], src_b[b], idx_sems[b])
        pltpu.async_copy(ei_hbm.at[pl.ds(E + base, CH)], dst_b[b],
                         idx_sems[b])

    def wait_idx(ic, b):
        base = ebase + ic * CH
        pltpu.make_async_copy(
            ei_hbm.at[pl.ds(base, CH)], src_b[b], idx_sems[b]
        ).wait()
        pltpu.make_async_copy(
            ei_hbm.at[pl.ds(E + base, CH)], dst_b[b], idx_sems[b]
        ).wait()

    def fire_data(ic, b):
        pltpu.async_copy(xh_hbm.at[src_b[b]], rows_v[b], in_sems[b])
        pltpu.async_copy(
            eh2_hbm.at[pl.ds(ebase + ic * CH, CH)], eh_v[b], in_sems[b]
        )

    def wait_data(ic, b):
        pltpu.make_async_copy(
            xh_hbm.at[src_b[b]], rows_v[b], in_sems[b]
        ).wait()
        pltpu.make_async_copy(
            eh2_hbm.at[pl.ds(ebase + ic * CH, CH)], eh_v[b], in_sems[b]
        ).wait()

    def compute(b):
        rv, ev = rows_v[b], eh_v[b]

        @plsc.parallel_loop(0, CH)
        def _(r):
            for k in range(8):
                sl = pl.ds(k * 16, 16)
                rv[r, sl] = jnp.maximum(rv[r, sl] + ev[r, sl], 0.0)

    def process(ic, b, nb):
        @pl.when(ic + 1 < NCHUNK)
        def _():
            wait_idx(ic + 1, nb)
            fire_data(ic + 1, nb)

        wait_data(ic, b)
        compute(b)
        d1 = pltpu.async_copy(rows_v[b], msum_sh.at[dst_b[b]], out_sem,
                              add=True)
        d2 = pltpu.async_copy(ones_v, deg_sh.at[dst_b[b]], out_sem,
                              add=True)
        d1.wait()
        d2.wait()

        @pl.when(ic + 2 < NCHUNK)
        def _():
            fire_idx(ic + 2, b)

    fire_idx(0, 0)
    fire_idx(1, 1)
    wait_idx(0, 0)
    fire_data(0, 0)

    @pl.loop(0, NCHUNK - 1, step=2)
    def _(i):
        process(i, 0, 1)
        process(i + 1, 1, 0)

    process(NCHUNK - 1, 0, 1)

    plsc.subcore_barrier()

    for t in range(5):
        sl = pl.ds(s * RPT + t * 128, 128)
        pltpu.async_copy(msum_sh.at[sl], msum_hbm.at[c, sl], in_sem0)
    for t in range(5):
        sl = pl.ds(s * RPT + t * 128, 128)
        pltpu.make_async_copy(msum_sh.at[sl], msum_hbm.at[c, sl],
                              in_sem0).wait()

    @pl.when(s == 0)
    def _():
        for t in range(5):
            pltpu.sync_copy(
                deg_sh.at[pl.ds(t * 2048, 2048)],
                deg_hbm.at[pl.ds(c * NPAD + t * 2048, 2048)],
            )


_sc_call = functools.partial(
    pl.kernel,
    out_type=(
        jax.ShapeDtypeStruct((NC, NPAD, D), jnp.float32),
        jax.ShapeDtypeStruct((NC * NPAD,), jnp.float32),
    ),
    mesh=plsc.VectorSubcoreMesh(
        core_axis_name="c", subcore_axis_name="s", num_cores=NC, num_subcores=NS
    ),
    scratch_types=[
        pltpu.VMEM((CH,), jnp.int32),
        pltpu.VMEM((CH,), jnp.int32),
        pltpu.VMEM((CH,), jnp.int32),
        pltpu.VMEM((CH,), jnp.int32),
        pltpu.VMEM((CH, D), jnp.float32),
        pltpu.VMEM((CH, D), jnp.float32),
        pltpu.VMEM((CH, D), jnp.float32),
        pltpu.VMEM((CH, D), jnp.float32),
        pltpu.VMEM((CH,), jnp.float32),
        pltpu.VMEM((2048,), jnp.float32),
        pltpu.VMEM_SHARED((NPAD, D), jnp.float32),
        pltpu.VMEM_SHARED((NPAD,), jnp.float32),
        pltpu.SemaphoreType.DMA,
        pltpu.SemaphoreType.DMA,
        pltpu.SemaphoreType.DMA,
        pltpu.SemaphoreType.DMA,
        pltpu.SemaphoreType.DMA,
    ],
)(_sc_body)


def _comb_body(p_ref, deg_ref, x_ref, ws_ref, bs_ref, o_ref):
    ms = p_ref[0] + p_ref[1]
    d = deg_ref[0] + deg_ref[1]
    r = 1.0 / jnp.maximum(d, 1.0)
    sf = (
        jnp.dot(x_ref[...], ws_ref[...], preferred_element_type=jnp.float32)
        + bs_ref[...][None, :]
    )
    o_ref[...] = ms * r + sf


def _combine(msum, deg, x, W_self, b_self):
    nb = 5
    rb = N // nb
    return pl.pallas_call(
        _comb_body,
        grid=(nb,),
        in_specs=[
            pl.BlockSpec((NC, rb, D), lambda i: (0, i, 0)),
            pl.BlockSpec((NC, rb, 1), lambda i: (0, i, 0)),
            pl.BlockSpec((rb, D), lambda i: (i, 0)),
            pl.BlockSpec((D, D), lambda i: (0, 0)),
            pl.BlockSpec((D,), lambda i: (0,)),
        ],
        out_specs=pl.BlockSpec((rb, D), lambda i: (i, 0)),
        out_shape=jax.ShapeDtypeStruct((N, D), jnp.float32),
    )(msum, deg, x, W_self, b_self)


def kernel(x, edge_index, edge_attr, W_edge, W_ne, b_ne, W_self, b_self):
    W1 = W_ne[:D, :]
    W2 = W_ne[D:, :]
    xh, W2e = _prep(x, W1, b_ne, W_edge, W2)
    eh2 = _eh2(edge_attr, W2e)
    ei_flat = edge_index.reshape(2 * E)
    msum, deg = _sc_call(xh, eh2, ei_flat)
    deg2 = deg.reshape(NC, NPAD, 1)
    return _combine(msum, deg2, x, W_self, b_self)
